# baseline (device time: 387593 ns/iter reference)
import functools

import jax
import jax.numpy as jnp
from jax import lax
from jax.experimental import pallas as pl
from jax.experimental.pallas import tpu as pltpu

N_Y = 4


def _mm_body(dy_ref, w_ref, out_ref):
    @pl.when(pl.program_id(0) == 0)
    def _():
        out_ref[...] = jnp.zeros_like(out_ref)

    out_ref[...] += lax.dot_general(
        dy_ref[...],
        w_ref[...],
        (((1,), (1,)), ((), ())),
        preferred_element_type=jnp.float32,
    )


def _ar_body(p_ref, out_ref, comm_ref, send_sems, recv_sems, credit_sem):
    my_x = lax.axis_index("x")
    my_y = lax.axis_index("y")
    my_z = lax.axis_index("z")
    left = (my_y - 1) % N_Y
    right = (my_y + 1) % N_Y

    chunk = out_ref.shape[0] // N_Y

    barrier_sem = pltpu.get_barrier_semaphore()
    for nbr in (left, right):
        pl.semaphore_signal(
            barrier_sem,
            inc=1,
            device_id=(my_x, nbr, my_z),
            device_id_type=pl.DeviceIdType.MESH,
        )
    pl.semaphore_wait(barrier_sem, 2)

    out_ref[...] = p_ref[...]

    for s in range(2 * (N_Y - 1)):
        if s < N_Y - 1:
            send_c = (my_y - s) % N_Y
        else:
            send_c = (my_y + N_Y - s) % N_Y
        recv_c = (send_c - 1) % N_Y
        slot = s % 2

        if s >= 2:
            pl.semaphore_wait(credit_sem, 1)

        rdma = pltpu.make_async_remote_copy(
            src_ref=out_ref.at[pl.ds(send_c * chunk, chunk), :],
            dst_ref=comm_ref.at[slot],
            send_sem=send_sems.at[slot],
            recv_sem=recv_sems.at[slot],
            device_id=(my_x, right, my_z),
            device_id_type=pl.DeviceIdType.MESH,
        )
        rdma.start()
        rdma.wait()

        if s < N_Y - 1:
            out_ref[pl.ds(recv_c * chunk, chunk), :] += comm_ref[slot]
        else:
            out_ref[pl.ds(recv_c * chunk, chunk), :] = comm_ref[slot]

        if s < 2 * (N_Y - 1) - 2:
            pl.semaphore_signal(
                credit_sem,
                inc=1,
                device_id=(my_x, left, my_z),
                device_id_type=pl.DeviceIdType.MESH,
            )


def kernel(dy, W):
    m, k_shard = dy.shape
    bk = 512

    partial = pl.pallas_call(
        _mm_body,
        grid=(k_shard // bk,),
        in_specs=[
            pl.BlockSpec((m, bk), lambda k: (0, k)),
            pl.BlockSpec((m, bk), lambda k: (0, k)),
        ],
        out_specs=pl.BlockSpec((m, m), lambda k: (0, 0)),
        out_shape=jax.ShapeDtypeStruct((m, m), jnp.float32),
    )(dy, W)

    chunk = m // N_Y
    return pl.pallas_call(
        _ar_body,
        out_shape=jax.ShapeDtypeStruct((m, m), jnp.float32),
        in_specs=[pl.BlockSpec(memory_space=pltpu.VMEM)],
        out_specs=pl.BlockSpec(memory_space=pltpu.VMEM),
        scratch_shapes=[
            pltpu.VMEM((2, chunk, m), jnp.float32),
            pltpu.SemaphoreType.DMA((2,)),
            pltpu.SemaphoreType.DMA((2,)),
            pltpu.SemaphoreType.REGULAR,
        ],
        compiler_params=pltpu.CompilerParams(collective_id=0),
    )(partial)


# device time: 273414 ns/iter; 1.4176x vs baseline; 1.4176x over previous
import functools

import jax
import jax.numpy as jnp
from jax import lax
from jax.experimental import pallas as pl
from jax.experimental.pallas import tpu as pltpu

N_Y = 4
M = 2048
HC = M // 2
CH = M // N_Y
MESH = pl.DeviceIdType.MESH


def _mm_body(dy_ref, w_ref, out_ref):
    @pl.when(pl.program_id(0) == 0)
    def _():
        out_ref[...] = jnp.zeros_like(out_ref)

    out_ref[...] += lax.dot_general(
        dy_ref[...],
        w_ref[...],
        (((1,), (1,)), ((), ())),
        preferred_element_type=jnp.float32,
    )


def _ar_body(p_ref, out_ref, acc, comm, send_sems, recv_sems, credit_sem,
             xland, xsend_s, xrecv_s):
    x_i = lax.axis_index("x")
    y_i = lax.axis_index("y")
    z_i = lax.axis_index("z")
    left = (y_i - 1) % N_Y
    right = (y_i + 1) % N_Y
    xp_dev = (1 - x_i, y_i, z_i)

    barrier_sem = pltpu.get_barrier_semaphore()
    for dev in ((x_i, left, z_i), (x_i, right, z_i), xp_dev):
        pl.semaphore_signal(
            barrier_sem, inc=1, device_id=dev, device_id_type=MESH
        )
    pl.semaphore_wait(barrier_sem, 3)

    acc[...] = p_ref[...]

    def xd(j, cc):
        return pltpu.make_async_remote_copy(
            src_ref=acc.at[pl.ds(cc * CH, CH), :],
            dst_ref=xland.at[j],
            send_sem=xsend_s.at[j],
            recv_sem=xrecv_s.at[j],
            device_id=xp_dev,
            device_id_type=MESH,
        )

    xinfo = []
    for s in range(2 * (N_Y - 1)):
        if s < N_Y - 1:
            send_c = (y_i - s) % N_Y
        else:
            send_c = (y_i + N_Y - s) % N_Y
        recv_c = (send_c - 1) % N_Y
        slot = s % 2

        if s >= 2:
            pl.semaphore_wait(credit_sem, 1)

        rdma = pltpu.make_async_remote_copy(
            src_ref=acc.at[pl.ds(send_c * CH, CH), :],
            dst_ref=comm.at[slot],
            send_sem=send_sems.at[slot],
            recv_sem=recv_sems.at[slot],
            device_id=(x_i, right, z_i),
            device_id_type=MESH,
        )
        rdma.start()
        rdma.wait()

        if s < N_Y - 1:
            acc[pl.ds(recv_c * CH, CH), :] += comm[slot]
        else:
            acc[pl.ds(recv_c * CH, CH), :] = comm[slot]

        if s < 2 * (N_Y - 1) - 2:
            pl.semaphore_signal(
                credit_sem,
                inc=1,
                device_id=(x_i, left, z_i),
                device_id_type=MESH,
            )

        if s >= 2:
            j = s - 2
            xd(j, recv_c).start()
            xinfo.append((j, recv_c))

    for j, cc in xinfo:
        d = xd(j, cc)
        d.wait_recv()
        rows = pl.ds(cc * CH, CH)

        @pl.when(x_i == 0)
        def _(rows=rows, j=j):
            out_ref[rows, 0:HC] = acc[rows, :]
            out_ref[rows, HC:M] = xland[j]

        @pl.when(x_i == 1)
        def _(rows=rows, j=j):
            out_ref[rows, HC:M] = acc[rows, :]
            out_ref[rows, 0:HC] = xland[j]

    for j, cc in xinfo:
        xd(j, cc).wait_send()

    @functools.partial(pl.run_scoped, exit_sem=pltpu.SemaphoreType.REGULAR)
    def _(exit_sem):
        for dev in ((x_i, left, z_i), (x_i, right, z_i), xp_dev):
            pl.semaphore_signal(
                exit_sem, inc=1, device_id=dev, device_id_type=MESH
            )
        pl.semaphore_wait(exit_sem, 3)


def kernel(dy, W):
    m, k_shard = dy.shape
    bk = 512

    x_i = lax.axis_index("x")
    w_half = lax.dynamic_slice(W, (x_i * HC, 0), (HC, k_shard))

    partial = pl.pallas_call(
        _mm_body,
        grid=(k_shard // bk,),
        in_specs=[
            pl.BlockSpec((m, bk), lambda k: (0, k)),
            pl.BlockSpec((HC, bk), lambda k: (0, k)),
        ],
        out_specs=pl.BlockSpec((m, HC), lambda k: (0, 0)),
        out_shape=jax.ShapeDtypeStruct((m, HC), jnp.float32),
    )(dy, w_half)

    return pl.pallas_call(
        _ar_body,
        out_shape=jax.ShapeDtypeStruct((m, m), jnp.float32),
        in_specs=[pl.BlockSpec(memory_space=pltpu.VMEM)],
        out_specs=pl.BlockSpec(memory_space=pltpu.VMEM),
        scratch_shapes=[
            pltpu.VMEM((m, HC), jnp.float32),
            pltpu.VMEM((2, CH, HC), jnp.float32),
            pltpu.SemaphoreType.DMA((2,)),
            pltpu.SemaphoreType.DMA((2,)),
            pltpu.SemaphoreType.REGULAR,
            pltpu.VMEM((N_Y, CH, HC), jnp.float32),
            pltpu.SemaphoreType.DMA((N_Y,)),
            pltpu.SemaphoreType.DMA((N_Y,)),
        ],
        compiler_params=pltpu.CompilerParams(
            collective_id=0, vmem_limit_bytes=56 * 1024 * 1024
        ),
    )(partial)


# device time: 273235 ns/iter; 1.4185x vs baseline; 1.0007x over previous
import functools

import jax
import jax.numpy as jnp
from jax import lax
from jax.experimental import pallas as pl
from jax.experimental.pallas import tpu as pltpu

N_Y = 4
M = 2048
HC = M // 2
CH = M // N_Y
MESH = pl.DeviceIdType.MESH


def _mm_body(dy_ref, w_ref, out_ref):
    @pl.when(pl.program_id(0) == 0)
    def _():
        out_ref[...] = jnp.zeros_like(out_ref)

    out_ref[...] += lax.dot_general(
        dy_ref[...].astype(jnp.bfloat16),
        w_ref[...].astype(jnp.bfloat16),
        (((1,), (1,)), ((), ())),
        preferred_element_type=jnp.float32,
    )


def _ar_body(p_ref, out_ref, acc, comm, send_sems, recv_sems, credit_sem,
             xland, xsend_s, xrecv_s):
    x_i = lax.axis_index("x")
    y_i = lax.axis_index("y")
    z_i = lax.axis_index("z")
    left = (y_i - 1) % N_Y
    right = (y_i + 1) % N_Y
    xp_dev = (1 - x_i, y_i, z_i)

    barrier_sem = pltpu.get_barrier_semaphore()
    for dev in ((x_i, left, z_i), (x_i, right, z_i), xp_dev):
        pl.semaphore_signal(
            barrier_sem, inc=1, device_id=dev, device_id_type=MESH
        )
    pl.semaphore_wait(barrier_sem, 3)

    acc[...] = p_ref[...]

    def xd(j, cc):
        return pltpu.make_async_remote_copy(
            src_ref=acc.at[pl.ds(cc * CH, CH), :],
            dst_ref=xland.at[j],
            send_sem=xsend_s.at[j],
            recv_sem=xrecv_s.at[j],
            device_id=xp_dev,
            device_id_type=MESH,
        )

    xinfo = []
    for s in range(2 * (N_Y - 1)):
        if s < N_Y - 1:
            send_c = (y_i - s) % N_Y
        else:
            send_c = (y_i + N_Y - s) % N_Y
        recv_c = (send_c - 1) % N_Y
        slot = s % 2

        if s >= 2:
            pl.semaphore_wait(credit_sem, 1)

        rdma = pltpu.make_async_remote_copy(
            src_ref=acc.at[pl.ds(send_c * CH, CH), :],
            dst_ref=comm.at[slot],
            send_sem=send_sems.at[slot],
            recv_sem=recv_sems.at[slot],
            device_id=(x_i, right, z_i),
            device_id_type=MESH,
        )
        rdma.start()
        rdma.wait()

        if s < N_Y - 1:
            acc[pl.ds(recv_c * CH, CH), :] += comm[slot]
        else:
            acc[pl.ds(recv_c * CH, CH), :] = comm[slot]

        if s < 2 * (N_Y - 1) - 2:
            pl.semaphore_signal(
                credit_sem,
                inc=1,
                device_id=(x_i, left, z_i),
                device_id_type=MESH,
            )

        if s >= 2:
            j = s - 2
            xd(j, recv_c).start()
            xinfo.append((j, recv_c))

    for j, cc in xinfo:
        d = xd(j, cc)
        d.wait_recv()
        rows = pl.ds(cc * CH, CH)

        @pl.when(x_i == 0)
        def _(rows=rows, j=j):
            out_ref[rows, 0:HC] = acc[rows, :]
            out_ref[rows, HC:M] = xland[j]

        @pl.when(x_i == 1)
        def _(rows=rows, j=j):
            out_ref[rows, HC:M] = acc[rows, :]
            out_ref[rows, 0:HC] = xland[j]

    for j, cc in xinfo:
        xd(j, cc).wait_send()

    @functools.partial(pl.run_scoped, exit_sem=pltpu.SemaphoreType.REGULAR)
    def _(exit_sem):
        for dev in ((x_i, left, z_i), (x_i, right, z_i), xp_dev):
            pl.semaphore_signal(
                exit_sem, inc=1, device_id=dev, device_id_type=MESH
            )
        pl.semaphore_wait(exit_sem, 3)


def kernel(dy, W):
    m, k_shard = dy.shape
    bk = 512

    x_i = lax.axis_index("x")
    w_half = lax.dynamic_slice(W, (x_i * HC, 0), (HC, k_shard))

    partial = pl.pallas_call(
        _mm_body,
        grid=(k_shard // bk,),
        in_specs=[
            pl.BlockSpec((m, bk), lambda k: (0, k)),
            pl.BlockSpec((HC, bk), lambda k: (0, k)),
        ],
        out_specs=pl.BlockSpec((m, HC), lambda k: (0, 0)),
        out_shape=jax.ShapeDtypeStruct((m, HC), jnp.float32),
    )(dy, w_half)

    return pl.pallas_call(
        _ar_body,
        out_shape=jax.ShapeDtypeStruct((m, m), jnp.float32),
        in_specs=[pl.BlockSpec(memory_space=pltpu.VMEM)],
        out_specs=pl.BlockSpec(memory_space=pltpu.VMEM),
        scratch_shapes=[
            pltpu.VMEM((m, HC), jnp.float32),
            pltpu.VMEM((2, CH, HC), jnp.float32),
            pltpu.SemaphoreType.DMA((2,)),
            pltpu.SemaphoreType.DMA((2,)),
            pltpu.SemaphoreType.REGULAR,
            pltpu.VMEM((N_Y, CH, HC), jnp.float32),
            pltpu.SemaphoreType.DMA((N_Y,)),
            pltpu.SemaphoreType.DMA((N_Y,)),
        ],
        compiler_params=pltpu.CompilerParams(
            collective_id=0, vmem_limit_bytes=56 * 1024 * 1024
        ),
    )(partial)
